# baseline (speedup 1.0000x reference)
import functools
import math

import jax
import jax.numpy as jnp
from jax import lax
from jax.experimental import pallas as pl
from jax.experimental.pallas import tpu as pltpu
from jax.experimental.pallas import tpu_sc as plsc

D = 64
SCALE = math.sqrt(D)
G = 128
K = 4
C = G * K
LANES = 16


def _make_sc_kernel(B: int, NC: int, NS: int):
  NW = NC * NS
  b_per_w = B // NW
  n_groups = b_per_w // G
  n_chunks = b_per_w // C
  assert b_per_w % C == 0 and n_chunks % 2 == 0 and n_chunks >= 4

  mesh = plsc.VectorSubcoreMesh(core_axis_name="c", subcore_axis_name="s")

  @functools.partial(
      pl.kernel,
      out_type=jax.ShapeDtypeStruct((B, D), jnp.float32),
      mesh=mesh,
      compiler_params=pltpu.CompilerParams(use_tc_tiling_on_sc=False),
      scratch_types=[
          pltpu.VMEM((n_groups, G), jnp.int32),
          pltpu.VMEM((C, D), jnp.float32),
          pltpu.VMEM((C, D), jnp.float32),
          pltpu.SemaphoreType.DMA,
          pltpu.SemaphoreType.DMA,
          pltpu.SemaphoreType.DMA,
          pltpu.SemaphoreType.DMA,
      ],
  )
  def k(idx_hbm, table_hbm, out_hbm, idx_v, buf0, buf1,
        gsem0, gsem1, osem0, osem1):
    cid = lax.axis_index("c")
    sid = lax.axis_index("s")
    wid = sid * NC + cid
    base = wid * b_per_w

    bufs = (buf0, buf1)
    gsems = (gsem0, gsem1)
    osems = (osem0, osem1)

    pltpu.sync_copy(idx_hbm.at[pl.ds(wid * n_groups, n_groups)], idx_v)

    def fire(c, b):
      for j in range(K):
        pltpu.async_copy(
            table_hbm.at[idx_v.at[c * K + j]],
            bufs[b].at[pl.ds(j * G, G)],
            gsems[b])

    def drain_gather(c, b):
      for j in range(K):
        pltpu.make_async_copy(
            table_hbm.at[idx_v.at[c * K + j]],
            bufs[b].at[pl.ds(j * G, G)],
            gsems[b]).wait()

    def scale(b):
      buf = bufs[b]

      @pl.loop(0, C, unroll=4)
      def _(i):
        for j in range(D // LANES):
          sl = (i, pl.ds(j * LANES, LANES))
          buf[sl] = buf[sl] * SCALE

    def write(c, b):
      pltpu.async_copy(bufs[b], out_hbm.at[pl.ds(base + c * C, C)], osems[b])

    def drain_write(c, b):
      pltpu.make_async_copy(
          bufs[b], out_hbm.at[pl.ds(base + c * C, C)], osems[b]).wait()

    fire(0, 0)
    fire(1, 1)
    drain_gather(0, 0)
    scale(0)
    write(0, 0)

    @pl.loop(1, n_chunks - 1, step=2)
    def _(c0):
      for b in range(2):
        c = c0 + b
        cur = (1 + b) % 2
        other = 1 - cur
        drain_write(c - 1, other)
        fire(c + 1, other)
        drain_gather(c, cur)
        scale(cur)
        write(c, cur)

    cl = n_chunks - 1
    drain_gather(cl, 1)
    scale(1)
    drain_write(cl - 1, 0)
    write(cl, 1)
    drain_write(cl, 1)

  return k


def kernel(indices, table):
  B0, S = indices.shape
  B = B0 * S
  info = plsc.get_sparse_core_info()
  NC, NS = info.num_cores, info.num_subcores
  idx2d = indices.astype(jnp.int32).reshape(B // G, G)
  out = _make_sc_kernel(B, NC, NS)(idx2d, table)
  return out.reshape(B0, S, D)
